# Initial kernel scaffold; baseline (speedup 1.0000x reference)
import functools

import jax
import jax.numpy as jnp
from jax import lax
from jax.experimental import pallas as pl
from jax.experimental.pallas import tpu as pltpu
from jax.experimental.pallas import tpu_sc as plsc

N = 10000
E = 320000
D = 128
H = 25
C = 40
HP = 32
CP = 48
NC, NS = 2, 16
NW = NC * NS
EPW = E // NW
CHUNK = 80
NCHUNK = EPW // CHUNK
DEG_SLAB = 640
NPAD = NS * DEG_SLAB
ROW_SLAB = N // NS

_MESH = plsc.VectorSubcoreMesh(
    core_axis_name="c", subcore_axis_name="s", num_cores=NC, num_subcores=NS)


def _deg_body(dst_hbm, ew_hbm, zero_hbm, out_hbm, dstv, eww, acc):
    core = lax.axis_index("c")
    tid = lax.axis_index("s")
    wid = core * NS + tid
    pltpu.sync_copy(dst_hbm.at[wid], dstv)
    pltpu.sync_copy(ew_hbm.at[wid], eww)
    slab = pl.ds(tid * DEG_SLAB, DEG_SLAB)
    pltpu.sync_copy(zero_hbm.at[slab], acc.at[slab])
    plsc.subcore_barrier()

    def body(c, carry):
        pltpu.sync_copy(eww.at[c], acc.at[dstv.at[c]], add=True)
        return carry

    lax.fori_loop(0, NCHUNK, body, 0)
    plsc.subcore_barrier()
    pltpu.sync_copy(acc.at[slab], out_hbm.at[core, slab])


_deg_call = pl.kernel(
    _deg_body,
    out_type=jax.ShapeDtypeStruct((NC, NPAD), jnp.float32),
    mesh=_MESH,
    scratch_types=[
        pltpu.VMEM((NCHUNK, CHUNK), jnp.int32),
        pltpu.VMEM((NCHUNK, CHUNK), jnp.float32),
        pltpu.VMEM_SHARED((NPAD,), jnp.float32),
    ],
)


def _agg_body(W, src_hbm, dst_hbm, ew_hbm, g_hbm, zero_hbm, out_hbm,
              srcv, dstv, eww, rows, acc, sem):
    core = lax.axis_index("c")
    tid = lax.axis_index("s")
    wid = core * NS + tid
    pltpu.sync_copy(src_hbm.at[wid], srcv)
    pltpu.sync_copy(dst_hbm.at[wid], dstv)
    pltpu.sync_copy(ew_hbm.at[wid], eww)
    slab = pl.ds(tid * ROW_SLAB, ROW_SLAB)
    pltpu.sync_copy(zero_hbm.at[slab], acc.at[slab])
    plsc.subcore_barrier()

    def body(c, carry):
        pltpu.async_copy(g_hbm.at[srcv.at[c]], rows, sem).wait()
        cvec = jnp.full((16,), c, jnp.int32)
        for e in range(CHUNK):
            coef = plsc.load_gather(eww, [cvec, jnp.full((16,), e, jnp.int32)])
            for v in range(W // 16):
                sl = pl.ds(v * 16, 16)
                rows[e, sl] = rows[e, sl] * coef
        pltpu.sync_copy(rows, acc.at[dstv.at[c]], add=True)
        return carry

    lax.fori_loop(0, NCHUNK, body, 0)
    plsc.subcore_barrier()
    pltpu.sync_copy(acc.at[slab], out_hbm.at[core, slab])


def _make_agg(W):
    return pl.kernel(
        functools.partial(_agg_body, W),
        out_type=jax.ShapeDtypeStruct((NC, N, W), jnp.float32),
        mesh=_MESH,
        scratch_types=[
            pltpu.VMEM((NCHUNK, CHUNK), jnp.int32),
            pltpu.VMEM((NCHUNK, CHUNK), jnp.int32),
            pltpu.VMEM((NCHUNK, CHUNK), jnp.float32),
            pltpu.VMEM((CHUNK, W), jnp.float32),
            pltpu.VMEM_SHARED((N, W), jnp.float32),
            pltpu.SemaphoreType.DMA,
        ],
    )


_agg_h = _make_agg(HP)
_agg_c = _make_agg(CP)


def _tc2_body(degp_ref, x_ref, w1_ref, g1_ref, dis_ref):
    deg = degp_ref[:, 0:1] + degp_ref[:, 1:2] + 1.0
    dis = lax.rsqrt(deg)
    h = jnp.dot(x_ref[:], w1_ref[:], preferred_element_type=jnp.float32)
    g1_ref[:] = h * dis
    dis_ref[:] = dis


_tc2 = pl.pallas_call(
    _tc2_body,
    out_shape=(
        jax.ShapeDtypeStruct((N, HP), jnp.float32),
        jax.ShapeDtypeStruct((N, 1), jnp.float32),
    ),
)


def _tc4_body(a_ref, g1_ref, dis_ref, b1_ref, w2_ref, g2_ref):
    s = a_ref[0] + a_ref[1] + g1_ref[:]
    z = jnp.maximum(dis_ref[:] * s + b1_ref[:], 0.0)
    h2 = jnp.dot(z, w2_ref[:], preferred_element_type=jnp.float32)
    g2_ref[:] = h2 * dis_ref[:]


_tc4 = pl.pallas_call(
    _tc4_body,
    out_shape=jax.ShapeDtypeStruct((N, CP), jnp.float32),
)


def _tc6_body(b_ref, g2_ref, dis_ref, b2_ref, out_ref):
    logits = dis_ref[:] * (b_ref[0] + b_ref[1] + g2_ref[:]) + b2_ref[:]
    l = logits[:, :C]
    m = jnp.max(l, axis=1, keepdims=True)
    s = jnp.sum(jnp.exp(l - m), axis=1, keepdims=True)
    out_ref[:] = l - m - jnp.log(s)


_tc6 = pl.pallas_call(
    _tc6_body,
    out_shape=jax.ShapeDtypeStruct((N, C), jnp.float32),
)


def kernel(x, edge_index, edge_weight, W1, b1, W2, b2):
    ei = edge_index.astype(jnp.int32)
    src = ei[0].reshape(NW, NCHUNK, CHUNK)
    dst = ei[1].reshape(NW, NCHUNK, CHUNK)
    ew = edge_weight.reshape(NW, NCHUNK, CHUNK)
    w1p = jnp.zeros((D, HP), jnp.float32).at[:, :H].set(W1)
    b1p = jnp.zeros((1, HP), jnp.float32).at[0, :H].set(b1)
    w2p = jnp.zeros((HP, CP), jnp.float32).at[:H, :C].set(W2)
    b2p = jnp.zeros((1, CP), jnp.float32).at[0, :C].set(b2)

    degp = _deg_call(dst, ew, jnp.zeros((NPAD,), jnp.float32))
    degp2 = degp[:, :N].T
    g1, dis = _tc2(degp2, x, w1p)
    a = _agg_h(src, dst, ew, g1, jnp.zeros((N, HP), jnp.float32))
    g2 = _tc4(a, g1, dis, b1p, w2p)
    b = _agg_c(src, dst, ew, g2, jnp.zeros((N, CP), jnp.float32))
    return _tc6(b, g2, dis, b2p)


# speedup vs baseline: 23.3361x; 23.3361x over previous
import functools

import jax
import jax.numpy as jnp
from jax import lax
from jax.experimental import pallas as pl
from jax.experimental.pallas import tpu as pltpu
from jax.experimental.pallas import tpu_sc as plsc

N = 10000
E = 320000
D = 128
H = 25
C = 40
HP = 32
CP = 48
NC, NS = 2, 16
NW = NC * NS
EPW = E // NW
CHUNK = 80
NCHUNK = EPW // CHUNK
DEG_SLAB = 640
NPAD = NS * DEG_SLAB
NR = NS * 640
ROW_SLAB = NR // NS

_MESH = plsc.VectorSubcoreMesh(
    core_axis_name="c", subcore_axis_name="s", num_cores=NC, num_subcores=NS)


def _deg_body(dst_hbm, ew_hbm, zero_hbm, out_hbm, dstv, eww, acc):
    core = lax.axis_index("c")
    tid = lax.axis_index("s")
    wid = core * NS + tid
    pltpu.sync_copy(dst_hbm.at[wid], dstv)
    pltpu.sync_copy(ew_hbm.at[wid], eww)
    slab = pl.ds(tid * DEG_SLAB, DEG_SLAB)
    pltpu.sync_copy(zero_hbm.at[slab], acc.at[slab])
    plsc.subcore_barrier()

    def body(c, carry):
        pltpu.sync_copy(eww.at[c], acc.at[dstv.at[c]], add=True)
        return carry

    lax.fori_loop(0, NCHUNK, body, 0)
    plsc.subcore_barrier()
    pltpu.sync_copy(acc.at[slab], out_hbm.at[core, slab])


_deg_call = pl.kernel(
    _deg_body,
    out_type=jax.ShapeDtypeStruct((NC, NPAD), jnp.float32),
    mesh=_MESH,
    scratch_types=[
        pltpu.VMEM((NCHUNK, CHUNK), jnp.int32),
        pltpu.VMEM((NCHUNK, CHUNK), jnp.float32),
        pltpu.VMEM_SHARED((NPAD,), jnp.float32),
    ],
)


def _agg_body(W, src_hbm, dst_hbm, ew_hbm, g_hbm, zero_hbm, out_hbm,
              srcv, dstv, eww, rows, acc, sem):
    core = lax.axis_index("c")
    tid = lax.axis_index("s")
    wid = core * NS + tid
    pltpu.sync_copy(src_hbm.at[wid], srcv)
    pltpu.sync_copy(dst_hbm.at[wid], dstv)
    pltpu.sync_copy(ew_hbm.at[wid], eww)
    slab = pl.ds(tid * ROW_SLAB, ROW_SLAB)
    pltpu.sync_copy(zero_hbm.at[slab], acc.at[slab])
    plsc.subcore_barrier()

    def body(c, carry):
        pltpu.async_copy(g_hbm.at[srcv.at[c]], rows, sem).wait()
        for grp in range(CHUNK // 16):
            vew = eww[pl.ds(c * CHUNK + grp * 16, 16)]
            for j in range(16):
                e = grp * 16 + j
                coef = lax.gather(
                    vew, jnp.full((16, 1), j, jnp.int32),
                    lax.GatherDimensionNumbers(
                        offset_dims=(), collapsed_slice_dims=(0,),
                        start_index_map=(0,)),
                    (1,), mode=lax.GatherScatterMode.PROMISE_IN_BOUNDS)
                for v in range(W // 16):
                    sl = pl.ds(v * 16, 16)
                    rows[e, sl] = rows[e, sl] * coef
        pltpu.sync_copy(rows, acc.at[dstv.at[c]], add=True)
        return carry

    lax.fori_loop(0, NCHUNK, body, 0)
    plsc.subcore_barrier()
    pltpu.sync_copy(acc.at[slab], out_hbm.at[core, slab])


def _make_agg(W):
    return pl.kernel(
        functools.partial(_agg_body, W),
        out_type=jax.ShapeDtypeStruct((NC, NR, W), jnp.float32),
        mesh=_MESH,
        compiler_params=pltpu.CompilerParams(use_tc_tiling_on_sc=False),
        scratch_types=[
            pltpu.VMEM((NCHUNK, CHUNK), jnp.int32),
            pltpu.VMEM((NCHUNK, CHUNK), jnp.int32),
            pltpu.VMEM((EPW,), jnp.float32),
            pltpu.VMEM((CHUNK, W), jnp.float32),
            pltpu.VMEM_SHARED((NR, W), jnp.float32),
            pltpu.SemaphoreType.DMA,
        ],
    )


_agg_h = _make_agg(HP)
_agg_c = _make_agg(CP)


def _tc2_body(degp_ref, x_ref, w1_ref, g1_ref, dis_ref):
    deg = degp_ref[:, 0:1] + degp_ref[:, 1:2] + 1.0
    dis = lax.rsqrt(deg)
    h = jnp.dot(x_ref[:], w1_ref[:], preferred_element_type=jnp.float32)
    g1_ref[:] = h * dis
    dis_ref[:] = dis


_tc2 = pl.pallas_call(
    _tc2_body,
    out_shape=(
        jax.ShapeDtypeStruct((N, HP), jnp.float32),
        jax.ShapeDtypeStruct((N, 1), jnp.float32),
    ),
)


def _tc4_body(a_ref, g1_ref, dis_ref, b1_ref, w2_ref, g2_ref):
    s = a_ref[0] + a_ref[1] + g1_ref[:]
    z = jnp.maximum(dis_ref[:] * s + b1_ref[:], 0.0)
    h2 = jnp.dot(z, w2_ref[:], preferred_element_type=jnp.float32)
    g2_ref[:] = h2 * dis_ref[:]


_tc4 = pl.pallas_call(
    _tc4_body,
    out_shape=jax.ShapeDtypeStruct((N, CP), jnp.float32),
)


def _tc6_body(b_ref, g2_ref, dis_ref, b2_ref, out_ref):
    logits = dis_ref[:] * (b_ref[0] + b_ref[1] + g2_ref[:]) + b2_ref[:]
    l = logits[:, :C]
    m = jnp.max(l, axis=1, keepdims=True)
    s = jnp.sum(jnp.exp(l - m), axis=1, keepdims=True)
    out_ref[:] = l - m - jnp.log(s)


_tc6 = pl.pallas_call(
    _tc6_body,
    out_shape=jax.ShapeDtypeStruct((N, C), jnp.float32),
)


def kernel(x, edge_index, edge_weight, W1, b1, W2, b2):
    ei = edge_index.astype(jnp.int32)
    src = ei[0].reshape(NW, NCHUNK, CHUNK)
    dst = ei[1].reshape(NW, NCHUNK, CHUNK)
    ew = edge_weight.reshape(NW, NCHUNK, CHUNK)
    ewf = edge_weight.reshape(NW, EPW)
    w1p = jnp.zeros((D, HP), jnp.float32).at[:, :H].set(W1)
    b1p = jnp.zeros((1, HP), jnp.float32).at[0, :H].set(b1)
    w2p = jnp.zeros((HP, CP), jnp.float32).at[:H, :C].set(W2)
    b2p = jnp.zeros((1, CP), jnp.float32).at[0, :C].set(b2)

    degp = _deg_call(dst, ew, jnp.zeros((NPAD,), jnp.float32))
    degp2 = degp[:, :N].T
    g1, dis = _tc2(degp2, x, w1p)
    a = _agg_h(src, dst, ewf, g1, jnp.zeros((NR, HP), jnp.float32))[:, :N]
    g2 = _tc4(a, g1, dis, b1p, w2p)
    b = _agg_c(src, dst, ewf, g2, jnp.zeros((NR, CP), jnp.float32))[:, :N]
    return _tc6(b, g2, dis, b2p)


# speedup vs baseline: 40.3394x; 1.7286x over previous
import functools

import jax
import jax.numpy as jnp
from jax import lax
from jax.experimental import pallas as pl
from jax.experimental.pallas import tpu as pltpu
from jax.experimental.pallas import tpu_sc as plsc

N = 10000
E = 320000
D = 128
H = 25
C = 40
HP = 32
CP = 48
NC, NS = 2, 16
NW = NC * NS
EPW = E // NW
CHUNK = 80
NCHUNK = EPW // CHUNK
DEG_SLAB = 640
NPAD = NS * DEG_SLAB
NR = NS * 640
ROW_SLAB = NR // NS

_MESH = plsc.VectorSubcoreMesh(
    core_axis_name="c", subcore_axis_name="s", num_cores=NC, num_subcores=NS)


def _deg_body(dst_hbm, ew_hbm, zero_hbm, out_hbm, dstv, eww, acc):
    core = lax.axis_index("c")
    tid = lax.axis_index("s")
    wid = core * NS + tid
    pltpu.sync_copy(dst_hbm.at[wid], dstv)
    pltpu.sync_copy(ew_hbm.at[wid], eww)
    slab = pl.ds(tid * DEG_SLAB, DEG_SLAB)
    pltpu.sync_copy(zero_hbm.at[slab], acc.at[slab])
    plsc.subcore_barrier()

    def body(c, carry):
        pltpu.sync_copy(eww.at[c], acc.at[dstv.at[c]], add=True)
        return carry

    lax.fori_loop(0, NCHUNK, body, 0)
    plsc.subcore_barrier()
    pltpu.sync_copy(acc.at[slab], out_hbm.at[core, slab])


_deg_call = pl.kernel(
    _deg_body,
    out_type=jax.ShapeDtypeStruct((NC, NPAD), jnp.float32),
    mesh=_MESH,
    scratch_types=[
        pltpu.VMEM((NCHUNK, CHUNK), jnp.int32),
        pltpu.VMEM((NCHUNK, CHUNK), jnp.float32),
        pltpu.VMEM_SHARED((NPAD,), jnp.float32),
    ],
)


NBUF = 5
OUTER = NCHUNK // NBUF


def _splat(vew, j):
    return lax.gather(
        vew, jnp.full((16, 1), j, jnp.int32),
        lax.GatherDimensionNumbers(
            offset_dims=(), collapsed_slice_dims=(0,), start_index_map=(0,)),
        (1,), mode=lax.GatherScatterMode.PROMISE_IN_BOUNDS)


def _agg_body(W, src_hbm, dst_hbm, ew_hbm, g_hbm, zero_hbm, out_hbm,
              srcv, dstv, eww, gb, pb, acc, *sems):
    gsem, ssem = sems[:NBUF], sems[NBUF:]
    core = lax.axis_index("c")
    tid = lax.axis_index("s")
    wid = core * NS + tid
    pltpu.sync_copy(src_hbm.at[wid], srcv)
    pltpu.sync_copy(dst_hbm.at[wid], dstv)
    pltpu.sync_copy(ew_hbm.at[wid], eww)
    slab = pl.ds(tid * ROW_SLAB, ROW_SLAB)
    pltpu.sync_copy(zero_hbm.at[slab], acc.at[slab])
    plsc.subcore_barrier()

    for b in range(NBUF):
        pltpu.async_copy(g_hbm.at[srcv.at[b]], gb.at[b], gsem[b])

    def step(c, b):
        pltpu.make_async_copy(g_hbm.at[srcv.at[c]], gb.at[b], gsem[b]).wait()
        for grp in range(CHUNK // 16):
            vew = eww[pl.ds(c * CHUNK + grp * 16, 16)]
            for j in range(16):
                e = grp * 16 + j
                coef = _splat(vew, j)
                for v in range(W // 16):
                    sl = pl.ds(v * 16, 16)
                    pb[b, e, sl] = gb[b, e, sl] * coef

        @pl.when(c + NBUF < NCHUNK)
        def _():
            pltpu.async_copy(g_hbm.at[srcv.at[c + NBUF]], gb.at[b], gsem[b])

        pltpu.async_copy(pb.at[b], acc.at[dstv.at[c]], ssem[b], add=True)

    for b in range(NBUF):
        step(b, b)

    def outer(i, carry):
        for b in range(NBUF):
            c = i * NBUF + b
            pltpu.make_async_copy(pb.at[b], acc.at[dstv.at[0]], ssem[b]).wait()
            step(c, b)
        return carry

    lax.fori_loop(1, OUTER, outer, 0)
    for b in range(NBUF):
        pltpu.make_async_copy(pb.at[b], acc.at[dstv.at[0]], ssem[b]).wait()
    plsc.subcore_barrier()
    pltpu.sync_copy(acc.at[slab], out_hbm.at[core, slab])


def _make_agg(W):
    return pl.kernel(
        functools.partial(_agg_body, W),
        out_type=jax.ShapeDtypeStruct((NC, NR, W), jnp.float32),
        mesh=_MESH,
        compiler_params=pltpu.CompilerParams(use_tc_tiling_on_sc=False),
        scratch_types=[
            pltpu.VMEM((NCHUNK, CHUNK), jnp.int32),
            pltpu.VMEM((NCHUNK, CHUNK), jnp.int32),
            pltpu.VMEM((EPW,), jnp.float32),
            pltpu.VMEM((NBUF, CHUNK, W), jnp.float32),
            pltpu.VMEM((NBUF, CHUNK, W), jnp.float32),
            pltpu.VMEM_SHARED((NR, W), jnp.float32),
        ] + [pltpu.SemaphoreType.DMA] * (2 * NBUF),
    )


_agg_h = _make_agg(HP)
_agg_c = _make_agg(CP)


def _tc2_body(degp_ref, x_ref, w1_ref, g1_ref, dis_ref):
    deg = degp_ref[:, 0:1] + degp_ref[:, 1:2] + 1.0
    dis = lax.rsqrt(deg)
    h = jnp.dot(x_ref[:], w1_ref[:], preferred_element_type=jnp.float32)
    g1_ref[:] = h * dis
    dis_ref[:] = dis


_tc2 = pl.pallas_call(
    _tc2_body,
    out_shape=(
        jax.ShapeDtypeStruct((N, HP), jnp.float32),
        jax.ShapeDtypeStruct((N, 1), jnp.float32),
    ),
)


def _tc4_body(a_ref, g1_ref, dis_ref, b1_ref, w2_ref, g2_ref):
    s = a_ref[0] + a_ref[1] + g1_ref[:]
    z = jnp.maximum(dis_ref[:] * s + b1_ref[:], 0.0)
    h2 = jnp.dot(z, w2_ref[:], preferred_element_type=jnp.float32)
    g2_ref[:] = h2 * dis_ref[:]


_tc4 = pl.pallas_call(
    _tc4_body,
    out_shape=jax.ShapeDtypeStruct((N, CP), jnp.float32),
)


def _tc6_body(b_ref, g2_ref, dis_ref, b2_ref, out_ref):
    logits = dis_ref[:] * (b_ref[0] + b_ref[1] + g2_ref[:]) + b2_ref[:]
    l = logits[:, :C]
    m = jnp.max(l, axis=1, keepdims=True)
    s = jnp.sum(jnp.exp(l - m), axis=1, keepdims=True)
    out_ref[:] = l - m - jnp.log(s)


_tc6 = pl.pallas_call(
    _tc6_body,
    out_shape=jax.ShapeDtypeStruct((N, C), jnp.float32),
)


def kernel(x, edge_index, edge_weight, W1, b1, W2, b2):
    ei = edge_index.astype(jnp.int32)
    src = ei[0].reshape(NW, NCHUNK, CHUNK)
    dst = ei[1].reshape(NW, NCHUNK, CHUNK)
    ew = edge_weight.reshape(NW, NCHUNK, CHUNK)
    ewf = edge_weight.reshape(NW, EPW)
    w1p = jnp.zeros((D, HP), jnp.float32).at[:, :H].set(W1)
    b1p = jnp.zeros((1, HP), jnp.float32).at[0, :H].set(b1)
    w2p = jnp.zeros((HP, CP), jnp.float32).at[:H, :C].set(W2)
    b2p = jnp.zeros((1, CP), jnp.float32).at[0, :C].set(b2)

    degp = _deg_call(dst, ew, jnp.zeros((NPAD,), jnp.float32))
    degp2 = degp[:, :N].T
    g1, dis = _tc2(degp2, x, w1p)
    a = _agg_h(src, dst, ewf, g1, jnp.zeros((NR, HP), jnp.float32))[:, :N]
    g2 = _tc4(a, g1, dis, b1p, w2p)
    b = _agg_c(src, dst, ewf, g2, jnp.zeros((NR, CP), jnp.float32))[:, :N]
    return _tc6(b, g2, dis, b2p)


# speedup vs baseline: 42.9250x; 1.0641x over previous
import functools

import jax
import jax.numpy as jnp
from jax import lax
from jax.experimental import pallas as pl
from jax.experimental.pallas import tpu as pltpu
from jax.experimental.pallas import tpu_sc as plsc

N = 10000
E = 320000
D = 128
H = 25
C = 40
HP = 32
CP = 48
NC, NS = 2, 16
NW = NC * NS
EPW = E // NW
CHUNK = 80
NCHUNK = EPW // CHUNK
DEG_SLAB = 640
NPAD = NS * DEG_SLAB
NR = NPAD
ROW_SLAB = NR // NS
NBUF = 5
OUTER = NCHUNK // NBUF
EPT = NC * EPW
NDCHUNK = EPT // CHUNK
DOUTER = NDCHUNK // NBUF

_MESH = plsc.VectorSubcoreMesh(
    core_axis_name="c", subcore_axis_name="s", num_cores=NC, num_subcores=NS)
_SC_PARAMS = pltpu.CompilerParams(use_tc_tiling_on_sc=False)


def _splat(vec, j):
    return lax.gather(
        vec, jnp.full((16, 1), j, jnp.int32),
        lax.GatherDimensionNumbers(
            offset_dims=(), collapsed_slice_dims=(0,), start_index_map=(0,)),
        (1,), mode=lax.GatherScatterMode.PROMISE_IN_BOUNDS)


def _fused_body(src_hbm, dst_hbm, ew_hbm, h1_hbm,
                zeros1_hbm, zerosh_hbm, outa_hbm, outdis_hbm,
                dstv2, eww2, srcv, dstv, eww, gb, pb, db, disv,
                acc1, dis_sp, acch, *sems):
    dsem = sems[0 * NBUF:1 * NBUF]
    gsem = sems[1 * NBUF:2 * NBUF]
    ssem = sems[2 * NBUF:3 * NBUF]
    esem = sems[3 * NBUF:4 * NBUF]
    core = lax.axis_index("c")
    tid = lax.axis_index("s")
    wid = core * NS + tid
    pltpu.sync_copy(dst_hbm.at[2 * tid], dstv2.at[pl.ds(0, NCHUNK)])
    pltpu.sync_copy(dst_hbm.at[2 * tid + 1], dstv2.at[pl.ds(NCHUNK, NCHUNK)])
    pltpu.sync_copy(ew_hbm.at[2 * tid], eww2.at[pl.ds(0, EPW)])
    pltpu.sync_copy(ew_hbm.at[2 * tid + 1], eww2.at[pl.ds(EPW, EPW)])
    pltpu.sync_copy(src_hbm.at[wid], srcv)
    pltpu.sync_copy(dst_hbm.at[wid], dstv)
    pltpu.sync_copy(ew_hbm.at[wid], eww)
    dslab = pl.ds(tid * DEG_SLAB, DEG_SLAB)
    rslab = pl.ds(tid * ROW_SLAB, ROW_SLAB)
    pltpu.sync_copy(zeros1_hbm.at[dslab], acc1.at[dslab])
    pltpu.sync_copy(zerosh_hbm.at[rslab], acch.at[rslab])
    plsc.subcore_barrier()

    def dstart(cc, b):
        pltpu.async_copy(eww2.at[pl.ds(cc * CHUNK, CHUNK)],
                         acc1.at[dstv2.at[cc]], dsem[b], add=True)

    def dwait(b):
        pltpu.make_async_copy(eww2.at[pl.ds(0, CHUNK)],
                              acc1.at[dstv2.at[0]], dsem[b]).wait()

    for b in range(NBUF):
        dstart(b, b)

    def douter(i, carry):
        for b in range(NBUF):
            cc = i * NBUF + b
            dwait(b)
            dstart(cc, b)
        return carry

    lax.fori_loop(1, DOUTER, douter, 0)
    for b in range(NBUF):
        dwait(b)
    plsc.subcore_barrier()

    pltpu.sync_copy(acc1.at[dslab], disv)
    for q in range(DEG_SLAB // 16):
        sl = pl.ds(q * 16, 16)
        d = disv[sl] + 1.0
        bits = lax.bitcast_convert_type(d, jnp.int32)
        y = lax.bitcast_convert_type(
            jnp.int32(0x5F3759DF) - lax.shift_right_logical(bits, 1),
            jnp.float32)
        for _ in range(3):
            y = y * (1.5 - 0.5 * d * y * y)
        disv[sl] = y
    pltpu.sync_copy(disv, dis_sp.at[dslab])
    pltpu.sync_copy(disv, outdis_hbm.at[core, dslab])
    plsc.subcore_barrier()

    for b in range(NBUF):
        pltpu.async_copy(h1_hbm.at[srcv.at[b]], gb.at[b], gsem[b])
        pltpu.async_copy(dis_sp.at[srcv.at[b]], db.at[b], esem[b])

    def step(c, b):
        pltpu.make_async_copy(h1_hbm.at[srcv.at[c]], gb.at[b], gsem[b]).wait()
        pltpu.make_async_copy(dis_sp.at[srcv.at[0]], db.at[b], esem[b]).wait()
        for grp in range(CHUNK // 16):
            vew = eww[pl.ds(c * CHUNK + grp * 16, 16)]
            vcoef = vew * db[b, pl.ds(grp * 16, 16)]
            for j in range(16):
                e = grp * 16 + j
                coef = _splat(vcoef, j)
                for v in range(HP // 16):
                    sl = pl.ds(v * 16, 16)
                    pb[b, e, sl] = gb[b, e, sl] * coef

        @pl.when(c + NBUF < NCHUNK)
        def _():
            pltpu.async_copy(h1_hbm.at[srcv.at[c + NBUF]], gb.at[b], gsem[b])
            pltpu.async_copy(dis_sp.at[srcv.at[c + NBUF]], db.at[b], esem[b])

        pltpu.async_copy(pb.at[b], acch.at[dstv.at[c]], ssem[b], add=True)

    for b in range(NBUF):
        step(b, b)

    def outer(i, carry):
        for b in range(NBUF):
            c = i * NBUF + b
            pltpu.make_async_copy(
                pb.at[b], acch.at[dstv.at[0]], ssem[b]).wait()
            step(c, b)
        return carry

    lax.fori_loop(1, OUTER, outer, 0)
    for b in range(NBUF):
        pltpu.make_async_copy(pb.at[b], acch.at[dstv.at[0]], ssem[b]).wait()
    plsc.subcore_barrier()
    pltpu.sync_copy(acch.at[rslab], outa_hbm.at[core, rslab])


_fused_call = pl.kernel(
    _fused_body,
    out_type=(
        jax.ShapeDtypeStruct((NC, NR, HP), jnp.float32),
        jax.ShapeDtypeStruct((NC, NPAD), jnp.float32),
    ),
    mesh=_MESH,
    compiler_params=_SC_PARAMS,
    scratch_types=[
        pltpu.VMEM((NDCHUNK, CHUNK), jnp.int32),
        pltpu.VMEM((EPT,), jnp.float32),
        pltpu.VMEM((NCHUNK, CHUNK), jnp.int32),
        pltpu.VMEM((NCHUNK, CHUNK), jnp.int32),
        pltpu.VMEM((EPW,), jnp.float32),
        pltpu.VMEM((NBUF, CHUNK, HP), jnp.float32),
        pltpu.VMEM((NBUF, CHUNK, HP), jnp.float32),
        pltpu.VMEM((NBUF, CHUNK), jnp.float32),
        pltpu.VMEM((DEG_SLAB,), jnp.float32),
        pltpu.VMEM_SHARED((NPAD,), jnp.float32),
        pltpu.VMEM_SHARED((NPAD,), jnp.float32),
        pltpu.VMEM_SHARED((NR, HP), jnp.float32),
    ] + [pltpu.SemaphoreType.DMA] * (4 * NBUF),
)


def _agg_body(W, src_hbm, dst_hbm, ew_hbm, g_hbm, zero_hbm, out_hbm,
              srcv, dstv, eww, gb, pb, acc, *sems):
    gsem, ssem = sems[:NBUF], sems[NBUF:]
    core = lax.axis_index("c")
    tid = lax.axis_index("s")
    wid = core * NS + tid
    pltpu.sync_copy(src_hbm.at[wid], srcv)
    pltpu.sync_copy(dst_hbm.at[wid], dstv)
    pltpu.sync_copy(ew_hbm.at[wid], eww)
    slab = pl.ds(tid * ROW_SLAB, ROW_SLAB)
    pltpu.sync_copy(zero_hbm.at[slab], acc.at[slab])
    plsc.subcore_barrier()

    for b in range(NBUF):
        pltpu.async_copy(g_hbm.at[srcv.at[b]], gb.at[b], gsem[b])

    def step(c, b):
        pltpu.make_async_copy(g_hbm.at[srcv.at[c]], gb.at[b], gsem[b]).wait()
        for grp in range(CHUNK // 16):
            vew = eww[pl.ds(c * CHUNK + grp * 16, 16)]
            for j in range(16):
                e = grp * 16 + j
                coef = _splat(vew, j)
                for v in range(W // 16):
                    sl = pl.ds(v * 16, 16)
                    pb[b, e, sl] = gb[b, e, sl] * coef

        @pl.when(c + NBUF < NCHUNK)
        def _():
            pltpu.async_copy(g_hbm.at[srcv.at[c + NBUF]], gb.at[b], gsem[b])

        pltpu.async_copy(pb.at[b], acc.at[dstv.at[c]], ssem[b], add=True)

    for b in range(NBUF):
        step(b, b)

    def outer(i, carry):
        for b in range(NBUF):
            c = i * NBUF + b
            pltpu.make_async_copy(pb.at[b], acc.at[dstv.at[0]], ssem[b]).wait()
            step(c, b)
        return carry

    lax.fori_loop(1, OUTER, outer, 0)
    for b in range(NBUF):
        pltpu.make_async_copy(pb.at[b], acc.at[dstv.at[0]], ssem[b]).wait()
    plsc.subcore_barrier()
    pltpu.sync_copy(acc.at[slab], out_hbm.at[core, slab])


def _make_agg(W):
    return pl.kernel(
        functools.partial(_agg_body, W),
        out_type=jax.ShapeDtypeStruct((NC, NR, W), jnp.float32),
        mesh=_MESH,
        compiler_params=_SC_PARAMS,
        scratch_types=[
            pltpu.VMEM((NCHUNK, CHUNK), jnp.int32),
            pltpu.VMEM((NCHUNK, CHUNK), jnp.int32),
            pltpu.VMEM((EPW,), jnp.float32),
            pltpu.VMEM((NBUF, CHUNK, W), jnp.float32),
            pltpu.VMEM((NBUF, CHUNK, W), jnp.float32),
            pltpu.VMEM_SHARED((NR, W), jnp.float32),
        ] + [pltpu.SemaphoreType.DMA] * (2 * NBUF),
    )


_agg_c = _make_agg(CP)


def _tc_mm_body(x_ref, w1_ref, h1_ref):
    h1_ref[:] = jnp.dot(x_ref[:], w1_ref[:],
                        preferred_element_type=jnp.float32)


_tc_mm = pl.pallas_call(
    _tc_mm_body,
    out_shape=jax.ShapeDtypeStruct((N, HP), jnp.float32),
)


def _tc_c_body(a_ref, h1_ref, dis_ref, b1_ref, w2_ref, g2_ref):
    dis = dis_ref[:]
    s = a_ref[0] + a_ref[1] + dis * h1_ref[:]
    z = jnp.maximum(dis * s + b1_ref[:], 0.0)
    h2 = jnp.dot(z, w2_ref[:], preferred_element_type=jnp.float32)
    g2_ref[:] = h2 * dis


_tc_c = pl.pallas_call(
    _tc_c_body,
    out_shape=jax.ShapeDtypeStruct((N, CP), jnp.float32),
)


def _tc_e_body(b_ref, g2_ref, dis_ref, b2_ref, out_ref):
    logits = dis_ref[:] * (b_ref[0] + b_ref[1] + g2_ref[:]) + b2_ref[:]
    l = logits[:, :C]
    m = jnp.max(l, axis=1, keepdims=True)
    s = jnp.sum(jnp.exp(l - m), axis=1, keepdims=True)
    out_ref[:] = l - m - jnp.log(s)


_tc_e = pl.pallas_call(
    _tc_e_body,
    out_shape=jax.ShapeDtypeStruct((N, C), jnp.float32),
)


def kernel(x, edge_index, edge_weight, W1, b1, W2, b2):
    ei = edge_index.astype(jnp.int32)
    src = ei[0].reshape(NW, NCHUNK, CHUNK)
    dst = ei[1].reshape(NW, NCHUNK, CHUNK)
    ewf = edge_weight.reshape(NW, EPW)
    w1p = jnp.zeros((D, HP), jnp.float32).at[:, :H].set(W1)
    b1p = jnp.zeros((1, HP), jnp.float32).at[0, :H].set(b1)
    w2p = jnp.zeros((HP, CP), jnp.float32).at[:H, :C].set(W2)
    b2p = jnp.zeros((1, CP), jnp.float32).at[0, :C].set(b2)

    h1 = _tc_mm(x, w1p)
    a, dis = _fused_call(src, dst, ewf, h1,
                         jnp.zeros((NPAD,), jnp.float32),
                         jnp.zeros((NR, HP), jnp.float32))
    dis2 = dis[0, :N].reshape(N, 1)
    g2 = _tc_c(a[:, :N], h1, dis2, b1p, w2p)
    b = _agg_c(src, dst, ewf, g2, jnp.zeros((NR, CP), jnp.float32))[:, :N]
    return _tc_e(b, g2, dis2, b2p)


# speedup vs baseline: 44.6666x; 1.0406x over previous
import functools

import jax
import jax.numpy as jnp
from jax import lax
from jax.experimental import pallas as pl
from jax.experimental.pallas import tpu as pltpu
from jax.experimental.pallas import tpu_sc as plsc

N = 10000
E = 320000
D = 128
H = 25
C = 40
HP = 32
CP = 48
NC, NS = 2, 16
NW = NC * NS
EPW = E // NW
BCH = 80
NROW = EPW // BCH
KROW = 5
CH = KROW * BCH
NCH = EPW // CH
NB = 3
DEG_SLAB = 640
NPAD = NS * DEG_SLAB
NR = NPAD
ROW_SLAB = NR // NS
NDROW = NC * NROW
NDCH = NDROW // KROW
DSEM = 5

_MESH = plsc.VectorSubcoreMesh(
    core_axis_name="c", subcore_axis_name="s", num_cores=NC, num_subcores=NS)
_SC_PARAMS = pltpu.CompilerParams(use_tc_tiling_on_sc=False)


def _splat(vec, j):
    return lax.gather(
        vec, jnp.full((16, 1), j, jnp.int32),
        lax.GatherDimensionNumbers(
            offset_dims=(), collapsed_slice_dims=(0,), start_index_map=(0,)),
        (1,), mode=lax.GatherScatterMode.PROMISE_IN_BOUNDS)


def _ewload(eww, c, grp):
    return eww[c * KROW + grp // (BCH // 16), pl.ds((grp % (BCH // 16)) * 16, 16)]


def _agg_loop(W, g_hbm, srcv, dstv, eww, gb, acc, gsem, ssem,
              db=None, dis_sp=None, esem=None):
    def gstart(c, b):
        idx = srcv.at[pl.ds(c * CH, CH)]
        pltpu.async_copy(g_hbm.at[idx], gb.at[b], gsem[b])
        if db is not None:
            pltpu.async_copy(dis_sp.at[idx], db.at[b], esem[b])

    def gwait(b):
        idx = srcv.at[pl.ds(0, CH)]
        pltpu.make_async_copy(g_hbm.at[idx], gb.at[b], gsem[b]).wait()
        if db is not None:
            pltpu.make_async_copy(dis_sp.at[idx], db.at[b], esem[b]).wait()

    def sstart(c, b):
        for r in range(KROW):
            pltpu.async_copy(gb.at[b, pl.ds(r * BCH, BCH)],
                             acc.at[dstv.at[c * KROW + r]], ssem[b], add=True)

    def swait(b):
        for _ in range(KROW):
            pltpu.make_async_copy(gb.at[b, pl.ds(0, BCH)],
                                  acc.at[dstv.at[0]], ssem[b]).wait()

    def step(c, b, tail, twait):
        gwait(b)

        def grp_body(grp, carry):
            vew = _ewload(eww, c, grp)
            if db is not None:
                vew = vew * db[b, pl.ds(grp * 16, 16)]
            for j in range(16):
                e = grp * 16 + j
                coef = _splat(vew, j)
                for v in range(W // 16):
                    sl = pl.ds(v * 16, 16)
                    gb[b, e, sl] = gb[b, e, sl] * coef
            return carry

        lax.fori_loop(0, CH // 16, grp_body, 0)
        sstart(c, b)
        if tail:
            tb = (b + 2) % NB
            if twait:
                swait(tb)
            gstart(c + 2, tb)

    gstart(0, 0)
    gstart(1, 1)
    step(0, 0, True, False)
    step(1, 1, True, True)

    def outer(i, carry):
        for j in range(NB):
            c = 2 + i * NB + j
            step(c, (2 + j) % NB, True, True)
        return carry

    lax.fori_loop(0, (NCH - 4) // NB, outer, 0)
    step(NCH - 2, (NCH - 2) % NB, False, False)
    step(NCH - 1, (NCH - 1) % NB, False, False)
    for cc in range(NCH - NB, NCH):
        swait(cc % NB)


def _fused_body(src_hbm, dst_hbm, ew_hbm, h1_hbm,
                zeros1_hbm, zerosh_hbm, outa_hbm, outdis_hbm,
                dstv2, eww2, srcv, dstv, eww, gb, db, disv,
                acc1, dis_sp, acch, *sems):
    dsem = sems[0 * DSEM:1 * DSEM]
    gsem = sems[DSEM + 0 * NB:DSEM + 1 * NB]
    ssem = sems[DSEM + 1 * NB:DSEM + 2 * NB]
    esem = sems[DSEM + 2 * NB:DSEM + 3 * NB]
    core = lax.axis_index("c")
    tid = lax.axis_index("s")
    wid = core * NS + tid
    pltpu.sync_copy(src_hbm.at[wid], srcv)
    pltpu.sync_copy(dst_hbm.at[wid], dstv)
    pltpu.sync_copy(ew_hbm.at[wid], eww)
    dslab = pl.ds(tid * DEG_SLAB, DEG_SLAB)
    rslab = pl.ds(tid * ROW_SLAB, ROW_SLAB)
    pltpu.sync_copy(zeros1_hbm.at[dslab], acc1.at[dslab])
    pltpu.sync_copy(zerosh_hbm.at[rslab], acch.at[rslab])
    plsc.subcore_barrier()

    def dstart(row, b):
        pltpu.async_copy(eww2.at[row], acc1.at[dstv2.at[row]],
                         dsem[b], add=True)

    def dwait(b):
        pltpu.make_async_copy(eww2.at[0], acc1.at[dstv2.at[0]],
                              dsem[b]).wait()

    def deg_half(slab_idx):
        pltpu.sync_copy(dst_hbm.at[slab_idx], dstv2)
        pltpu.sync_copy(ew_hbm.at[slab_idx], eww2)
        for b in range(DSEM):
            dstart(b, b)

        def douter(i, carry):
            for b in range(DSEM):
                dwait(b)
                dstart(DSEM + i * DSEM + b, b)
            return carry

        lax.fori_loop(0, NROW // DSEM - 1, douter, 0)
        for b in range(DSEM):
            dwait(b)

    deg_half(2 * tid)
    deg_half(2 * tid + 1)
    plsc.subcore_barrier()

    pltpu.sync_copy(acc1.at[dslab], disv)
    for q in range(DEG_SLAB // 16):
        sl = pl.ds(q * 16, 16)
        d = disv[sl] + 1.0
        bits = lax.bitcast_convert_type(d, jnp.int32)
        y = lax.bitcast_convert_type(
            jnp.int32(0x5F3759DF) - lax.shift_right_logical(bits, 1),
            jnp.float32)
        for _ in range(3):
            y = y * (1.5 - 0.5 * d * y * y)
        disv[sl] = y
    pltpu.sync_copy(disv, dis_sp.at[dslab])
    pltpu.sync_copy(disv, outdis_hbm.at[core, dslab])
    plsc.subcore_barrier()

    _agg_loop(HP, h1_hbm, srcv, dstv, eww, gb, acch, gsem, ssem,
              db=db, dis_sp=dis_sp, esem=esem)
    plsc.subcore_barrier()
    pltpu.sync_copy(acch.at[rslab], outa_hbm.at[core, rslab])


_fused_call = pl.kernel(
    _fused_body,
    out_type=(
        jax.ShapeDtypeStruct((NC, NR, HP), jnp.float32),
        jax.ShapeDtypeStruct((NC, NPAD), jnp.float32),
    ),
    mesh=_MESH,
    compiler_params=_SC_PARAMS,
    scratch_types=[
        pltpu.VMEM((NROW, BCH), jnp.int32),
        pltpu.VMEM((NROW, BCH), jnp.float32),
        pltpu.VMEM((EPW,), jnp.int32),
        pltpu.VMEM((NROW, BCH), jnp.int32),
        pltpu.VMEM((NROW, BCH), jnp.float32),
        pltpu.VMEM((NB, CH, HP), jnp.float32),
        pltpu.VMEM((NB, CH), jnp.float32),
        pltpu.VMEM((DEG_SLAB,), jnp.float32),
        pltpu.VMEM_SHARED((NPAD,), jnp.float32),
        pltpu.VMEM_SHARED((NPAD,), jnp.float32),
        pltpu.VMEM_SHARED((NR, HP), jnp.float32),
    ] + [pltpu.SemaphoreType.DMA] * (DSEM + 3 * NB),
)


def _agg_body(W, src_hbm, dst_hbm, ew_hbm, g_hbm, zero_hbm, out_hbm,
              srcv, dstv, eww, gb, acc, *sems):
    gsem, ssem = sems[:NB], sems[NB:]
    core = lax.axis_index("c")
    tid = lax.axis_index("s")
    wid = core * NS + tid
    pltpu.sync_copy(src_hbm.at[wid], srcv)
    pltpu.sync_copy(dst_hbm.at[wid], dstv)
    pltpu.sync_copy(ew_hbm.at[wid], eww)
    slab = pl.ds(tid * ROW_SLAB, ROW_SLAB)
    pltpu.sync_copy(zero_hbm.at[slab], acc.at[slab])
    plsc.subcore_barrier()
    _agg_loop(W, g_hbm, srcv, dstv, eww, gb, acc, gsem, ssem)
    plsc.subcore_barrier()
    pltpu.sync_copy(acc.at[slab], out_hbm.at[core, slab])


def _make_agg(W):
    return pl.kernel(
        functools.partial(_agg_body, W),
        out_type=jax.ShapeDtypeStruct((NC, NR, W), jnp.float32),
        mesh=_MESH,
        compiler_params=_SC_PARAMS,
        scratch_types=[
            pltpu.VMEM((EPW,), jnp.int32),
            pltpu.VMEM((NROW, BCH), jnp.int32),
            pltpu.VMEM((NROW, BCH), jnp.float32),
            pltpu.VMEM((NB, CH, W), jnp.float32),
            pltpu.VMEM_SHARED((NR, W), jnp.float32),
        ] + [pltpu.SemaphoreType.DMA] * (2 * NB),
    )


_agg_c = _make_agg(CP)


def _tc_mm_body(x_ref, w1_ref, h1_ref):
    h1_ref[:] = jnp.dot(x_ref[:], w1_ref[:],
                        preferred_element_type=jnp.float32)


_tc_mm = pl.pallas_call(
    _tc_mm_body,
    out_shape=jax.ShapeDtypeStruct((N, HP), jnp.float32),
)


def _tc_c_body(a_ref, h1_ref, dis_ref, b1_ref, w2_ref, g2_ref):
    dis = dis_ref[:]
    s = a_ref[0] + a_ref[1] + dis * h1_ref[:]
    z = jnp.maximum(dis * s + b1_ref[:], 0.0)
    h2 = jnp.dot(z, w2_ref[:], preferred_element_type=jnp.float32)
    g2_ref[:] = h2 * dis


_tc_c = pl.pallas_call(
    _tc_c_body,
    out_shape=jax.ShapeDtypeStruct((N, CP), jnp.float32),
)


def _tc_e_body(b_ref, g2_ref, dis_ref, b2_ref, out_ref):
    logits = dis_ref[:] * (b_ref[0] + b_ref[1] + g2_ref[:]) + b2_ref[:]
    l = logits[:, :C]
    m = jnp.max(l, axis=1, keepdims=True)
    s = jnp.sum(jnp.exp(l - m), axis=1, keepdims=True)
    out_ref[:] = l - m - jnp.log(s)


_tc_e = pl.pallas_call(
    _tc_e_body,
    out_shape=jax.ShapeDtypeStruct((N, C), jnp.float32),
)


def kernel(x, edge_index, edge_weight, W1, b1, W2, b2):
    ei = edge_index.astype(jnp.int32)
    src = ei[0].reshape(NW, EPW)
    dst = ei[1].reshape(NW, NROW, BCH)
    ew = edge_weight.reshape(NW, NROW, BCH)
    w1p = jnp.zeros((D, HP), jnp.float32).at[:, :H].set(W1)
    b1p = jnp.zeros((1, HP), jnp.float32).at[0, :H].set(b1)
    w2p = jnp.zeros((HP, CP), jnp.float32).at[:H, :C].set(W2)
    b2p = jnp.zeros((1, CP), jnp.float32).at[0, :C].set(b2)

    h1 = _tc_mm(x, w1p)
    a, dis = _fused_call(src, dst, ew, h1,
                         jnp.zeros((NPAD,), jnp.float32),
                         jnp.zeros((NR, HP), jnp.float32))
    dis2 = dis[0, :N].reshape(N, 1)
    g2 = _tc_c(a[:, :N], h1, dis2, b1p, w2p)
    b = _agg_c(src, dst, ew, g2, jnp.zeros((NR, CP), jnp.float32))[:, :N]
    return _tc_e(b, g2, dis2, b2p)
